# baseline (speedup 1.0000x reference)
import functools

import jax
import jax.numpy as jnp
from jax import lax
from jax.experimental import pallas as pl
from jax.experimental.pallas import tpu as pltpu
from jax.experimental.pallas import tpu_sc as plsc

VOCAB = 1000000
LENGTH = 200
DIM = 64
BATCH = 4096


def _positional_encoding(length, dim, n=10000):
    half_dim = dim // 2
    pos = jnp.arange(length, dtype=jnp.float32).reshape(-1, 1)
    i = jnp.arange(half_dim, dtype=jnp.float32).reshape(1, -1)
    denom = jnp.power(jnp.float32(n), -i / half_dim)
    args = pos * denom
    sin = jnp.expand_dims(jnp.sin(args), axis=-1)
    cos = jnp.expand_dims(jnp.cos(args), axis=-1)
    return jnp.concatenate([sin, cos], axis=-1).reshape(length, dim)


def _make_sc_kernel(num_cores, num_subcores):
    nw = num_cores * num_subcores
    bw = BATCH // nw
    nbt = BATCH // 128
    mesh = plsc.VectorSubcoreMesh(core_axis_name="c", subcore_axis_name="s")

    @functools.partial(
        pl.kernel,
        mesh=mesh,
        out_type=jax.ShapeDtypeStruct((LENGTH, DIM // 8, nbt, 8, 128), jnp.float32),
        scratch_types=[
            pltpu.VMEM((LENGTH, bw), jnp.int32),
            pltpu.VMEM((LENGTH, DIM), jnp.float32),
            pltpu.VMEM((2, bw, 128), jnp.float32),
            pltpu.VMEM((2, DIM // 8, 8, 128), jnp.float32),
            pltpu.SemaphoreType.DMA,
            pltpu.SemaphoreType.DMA,
            pltpu.SemaphoreType.DMA,
        ],
        compiler_params=pltpu.CompilerParams(needs_layout_passes=False),
    )
    def sc_kernel(idx_hbm, table_hbm, pe_hbm, out_hbm,
                  idx_v, pe_v, rows_v, blk_v, gsem0, gsem1, osem):
        wid = lax.axis_index("s") * num_cores + lax.axis_index("c")
        b0 = wid * bw
        pltpu.sync_copy(idx_hbm.at[:, pl.ds(b0, bw)], idx_v)
        pltpu.sync_copy(pe_hbm, pe_v)

        gsems = (gsem0, gsem1)

        def gather_copy(p, slot):
            return pltpu.make_async_copy(
                table_hbm.at[idx_v.at[p]],
                rows_v.at[slot],
                gsems[slot],
            )

        def block_write(p, slot):
            return pltpu.make_async_copy(
                blk_v.at[slot],
                out_hbm.at[p, :, wid],
                osem,
            )

        def compute(p, slot):
            ci = lax.iota(jnp.int32, 16)

            @pl.loop(0, DIM // 8)
            def _dt(dt):
                for r in range(8):
                    d = dt * 8 + r
                    dsplat = jnp.full((16,), d, jnp.int32)
                    psplat = plsc.load_gather(pe_v.at[p], [dsplat])
                    for cg in range(8):
                        vals = plsc.load_gather(
                            rows_v.at[slot], [ci + (cg * 16), dsplat]
                        )
                        blk_v[slot, dt, r, pl.ds(cg * 16, 16)] = vals + psplat

        gather_copy(0, 0).start()

        @pl.loop(0, LENGTH // 2)
        def _pos(g):
            for half in range(2):
                p = g * 2 + half
                slot = half
                @pl.when(p + 1 < LENGTH)
                def _():
                    gather_copy(p + 1, 1 - slot).start()

                gather_copy(p, slot).wait()

                @pl.when(p >= 2)
                def _():
                    block_write(p - 2, slot).wait()

                compute(p, slot)
                block_write(p, slot).start()

        block_write(LENGTH - 2, 0).wait()
        block_write(LENGTH - 1, 1).wait()

    return sc_kernel


def kernel(inputs, table):
    pe = _positional_encoding(LENGTH, DIM)
    info = plsc.get_sparse_core_info()
    sc_kernel = _make_sc_kernel(info.num_cores, info.num_subcores)
    idx_t = jnp.transpose(inputs.astype(jnp.int32))
    t128 = jnp.pad(table, ((0, 0), (0, DIM)))
    out5 = sc_kernel(idx_t, t128, pe)
    return out5.transpose(2, 4, 0, 1, 3).reshape(BATCH, LENGTH, DIM)
